# Initial kernel scaffold; baseline (speedup 1.0000x reference)
import jax
import jax.numpy as jnp
from jax.experimental import pallas as pl


def kernel(nf, ef, edge_index, We1, be1, We2, be2, We3, be3, Wn1, bn1, Wn2, bn2, Wn3, bn3):
    raise NotImplementedError("write your pallas kernel here")


# speedup vs baseline: 3.5855x; 3.5855x over previous
import functools

import jax
import jax.numpy as jnp
from jax import lax
from jax.experimental import pallas as pl
from jax.experimental.pallas import tpu as pltpu
from jax.experimental.pallas import tpu_sc as plsc

N = 10000
E = 320000
D_NODE = 128
D_EDGE = 16
D_EOUT = 16
D_NOUT = 128
H = 64

NC = 2
NS = 16
NW = NC * NS
EPW = E // NW
CHUNK = 80
NCHUNK = EPW // CHUNK
NPAD = 10240
NPW = NPAD // NS

_mesh = plsc.VectorSubcoreMesh(
    core_axis_name="c", subcore_axis_name="s", num_cores=NC, num_subcores=NS
)


def _proj_body(nf_ref, w_ref, t_ref):
    t_ref[...] = jnp.dot(nf_ref[...], w_ref[...],
                         preferred_element_type=jnp.float32)


def _proj(nf, w):
    bn = 2000
    return pl.pallas_call(
        _proj_body,
        grid=(N // bn,),
        in_specs=[
            pl.BlockSpec((bn, D_NODE), lambda i: (i, 0)),
            pl.BlockSpec((D_NODE, 2 * H), lambda i: (0, 0)),
        ],
        out_specs=pl.BlockSpec((bn, 2 * H), lambda i: (i, 0)),
        out_shape=jax.ShapeDtypeStruct((N, 2 * H), jnp.float32),
    )(nf, w)


@functools.partial(
    pl.kernel,
    out_type=jax.ShapeDtypeStruct((E, H), jnp.float32),
    mesh=_mesh,
    scratch_types=[
        pltpu.VMEM((NCHUNK, CHUNK), jnp.int32),
        pltpu.VMEM((NCHUNK, CHUNK), jnp.int32),
        pltpu.VMEM((CHUNK, 2 * H), jnp.float32),
        pltpu.VMEM((CHUNK, 2 * H), jnp.float32),
        pltpu.VMEM((CHUNK, H), jnp.float32),
        pltpu.SemaphoreType.DMA,
        pltpu.SemaphoreType.DMA,
    ],
)
def _gather_kernel(t_hbm, src_hbm, dst_hbm, g_hbm,
                   src_v, dst_v, rows_s, rows_d, gbuf, sem_s, sem_d):
    cid = lax.axis_index("c")
    sid = lax.axis_index("s")
    wid = sid * NC + cid
    pltpu.sync_copy(src_hbm.at[wid], src_v)
    pltpu.sync_copy(dst_hbm.at[wid], dst_v)

    def chunk_body(c, carry):
        eoff = wid * EPW + c * CHUNK
        cp_s = pltpu.async_copy(t_hbm.at[src_v.at[c]], rows_s, sem_s)
        cp_d = pltpu.async_copy(t_hbm.at[dst_v.at[c]], rows_d, sem_d)
        cp_s.wait()
        cp_d.wait()

        def add_row(i, acc):
            for j in range(H // 16):
                sl = pl.ds(j * 16, 16)
                sl_hi = pl.ds(H + j * 16, 16)
                gbuf[i, sl] = rows_s[i, sl] + rows_d[i, sl_hi]
            return acc

        lax.fori_loop(0, CHUNK, add_row, 0)
        pltpu.sync_copy(gbuf, g_hbm.at[pl.ds(eoff, CHUNK)])
        return carry

    lax.fori_loop(0, NCHUNK, chunk_body, 0)


def _edge_mlp_body(ef_ref, g_ref, w1_ref, b1_ref, w2_ref, b2_ref,
                   w3_ref, b3_ref, out_ref):
    h1 = jnp.dot(ef_ref[...], w1_ref[...], preferred_element_type=jnp.float32)
    h1 = jnp.maximum(h1 + g_ref[...] + b1_ref[...], 0.0)
    h2 = jnp.dot(h1, w2_ref[...], preferred_element_type=jnp.float32)
    h2 = jnp.maximum(h2 + b2_ref[...], 0.0)
    h3 = jnp.dot(h2, w3_ref[...], preferred_element_type=jnp.float32)
    out_ref[...] = jnp.maximum(h3 + b3_ref[...], 0.0)


def _edge_mlp(ef, g, w1, b1, w2, b2, w3, b3):
    be = 3200
    wspec = lambda r, c: pl.BlockSpec((r, c), lambda i: (0, 0))
    return pl.pallas_call(
        _edge_mlp_body,
        grid=(E // be,),
        in_specs=[
            pl.BlockSpec((be, D_EDGE), lambda i: (i, 0)),
            pl.BlockSpec((be, H), lambda i: (i, 0)),
            wspec(D_EDGE, H), wspec(1, H),
            wspec(H, H), wspec(1, H),
            wspec(H, D_EOUT), wspec(1, D_EOUT),
        ],
        out_specs=pl.BlockSpec((be, D_EOUT), lambda i: (i, 0)),
        out_shape=jax.ShapeDtypeStruct((E, D_EOUT), jnp.float32),
    )(ef, g, w1, b1, w2, b2, w3, b3)


NBLK = 5
CPB = 250
FPB = CPB * 128
SLICE = NPAD * D_EOUT // NS


@functools.partial(
    pl.kernel,
    out_type=jax.ShapeDtypeStruct((NC, NPAD * D_EOUT), jnp.float32),
    mesh=_mesh,
    scratch_types=[
        pltpu.VMEM((FPB,), jnp.float32),
        pltpu.VMEM((CPB, 128), jnp.int32),
        pltpu.VMEM((SLICE,), jnp.float32),
        pltpu.VMEM_SHARED((NPAD * D_EOUT,), jnp.float32),
        pltpu.SemaphoreType.DMA,
    ],
)
def _scatter_kernel(uef_hbm, idx_hbm, out_hbm, data_v, idx_v, stage_v,
                    agg_sh, sem):
    cid = lax.axis_index("c")
    sid = lax.axis_index("s")
    wid = sid * NC + cid

    def zrow(i, carry):
        stage_v[pl.ds(i * 16, 16)] = jnp.zeros((16,), jnp.float32)
        return carry

    lax.fori_loop(0, SLICE // 16, zrow, 0)
    pltpu.sync_copy(stage_v, agg_sh.at[pl.ds(sid * SLICE, SLICE)])
    plsc.subcore_barrier()

    def blk(b, carry):
        foff = wid * (EPW * D_EOUT) + b * FPB
        pltpu.sync_copy(uef_hbm.at[pl.ds(foff, FPB)], data_v)
        pltpu.sync_copy(idx_hbm.at[wid].at[b], idx_v)

        def chnk(k, c2):
            pltpu.async_copy(data_v.at[pl.ds(k * 128, 128)],
                             agg_sh.at[idx_v.at[k]], sem, add=True)
            return c2

        lax.fori_loop(0, CPB, chnk, 0)
        pltpu.make_async_copy(uef_hbm.at[pl.ds(foff, FPB)], data_v, sem).wait()
        return carry

    lax.fori_loop(0, NBLK, blk, 0)
    plsc.subcore_barrier()

    pltpu.sync_copy(agg_sh.at[pl.ds(sid * SLICE, SLICE)], stage_v)
    pltpu.sync_copy(stage_v, out_hbm.at[cid].at[pl.ds(sid * SLICE, SLICE)])


def _node_mlp_body(pp_ref, nf_ref, wa_ref, wb_ref, b1_ref, w2_ref, b2_ref,
                   w3_ref, b3_ref, out_ref):
    agg = pp_ref[0] + pp_ref[1]
    h1 = (jnp.dot(agg, wa_ref[...], preferred_element_type=jnp.float32)
          + jnp.dot(nf_ref[...], wb_ref[...], preferred_element_type=jnp.float32))
    h1 = jnp.maximum(h1 + b1_ref[...], 0.0)
    h2 = jnp.dot(h1, w2_ref[...], preferred_element_type=jnp.float32)
    h2 = jnp.maximum(h2 + b2_ref[...], 0.0)
    h3 = jnp.dot(h2, w3_ref[...], preferred_element_type=jnp.float32)
    out_ref[...] = jnp.maximum(h3 + b3_ref[...], 0.0)


def _node_mlp(pp, nf, wa, wb, b1, w2, b2, w3, b3):
    bn = 2000
    wspec = lambda r, c: pl.BlockSpec((r, c), lambda i: (0, 0))
    return pl.pallas_call(
        _node_mlp_body,
        grid=(N // bn,),
        in_specs=[
            pl.BlockSpec((NC, bn, D_EOUT), lambda i: (0, i, 0)),
            pl.BlockSpec((bn, D_NODE), lambda i: (i, 0)),
            wspec(D_EOUT, H), wspec(D_NODE, H), wspec(1, H),
            wspec(H, H), wspec(1, H),
            wspec(H, D_NOUT), wspec(1, D_NOUT),
        ],
        out_specs=pl.BlockSpec((bn, D_NOUT), lambda i: (i, 0)),
        out_shape=jax.ShapeDtypeStruct((N, D_NOUT), jnp.float32),
    )(pp, nf, wa, wb, b1, w2, b2, w3, b3)


def kernel(nf, ef, edge_index, We1, be1, We2, be2, We3, be3,
           Wn1, bn1, Wn2, bn2, Wn3, bn3):
    src = edge_index[0].astype(jnp.int32).reshape(NW, NCHUNK, CHUNK)
    dst = edge_index[1].astype(jnp.int32).reshape(NW, NCHUNK, CHUNK)
    we1e = We1[:D_EDGE]
    we1sd = jnp.concatenate([We1[D_EDGE:D_EDGE + D_NODE],
                             We1[D_EDGE + D_NODE:]], axis=1)
    wn1a = Wn1[:D_EOUT]
    wn1b = Wn1[D_EOUT:]

    t = _proj(nf, we1sd)
    g = _gather_kernel(t, src, dst)
    u_ef = _edge_mlp(ef, g, we1e, be1.reshape(1, -1), We2, be2.reshape(1, -1),
                     We3, be3.reshape(1, -1))
    dst_flat = edge_index[1].astype(jnp.int32)
    idx_elem = ((dst_flat * D_EOUT)[:, None]
                + jnp.arange(D_EOUT, dtype=jnp.int32)[None, :])
    idx_elem = idx_elem.reshape(NW, NBLK, CPB, 128)
    uef_flat = u_ef.reshape(E * D_EOUT)
    partials = _scatter_kernel(uef_flat, idx_elem)
    partials = partials.reshape(NC, NPAD, D_EOUT)
    u_nf = _node_mlp(partials, nf, wn1a, wn1b, bn1.reshape(1, -1),
                     Wn2, bn2.reshape(1, -1), Wn3, bn3.reshape(1, -1))
    return (u_nf, u_ef)


# speedup vs baseline: 3.8590x; 1.0763x over previous
import functools

import jax
import jax.numpy as jnp
from jax import lax
from jax.experimental import pallas as pl
from jax.experimental.pallas import tpu as pltpu
from jax.experimental.pallas import tpu_sc as plsc

N = 10000
E = 320000
D_NODE = 128
D_EDGE = 16
D_EOUT = 16
D_NOUT = 128
H = 64

NC = 2
NS = 16
NW = NC * NS
EPW = E // NW
CHUNK = 40
NCHUNK = EPW // CHUNK
NPAD = 10240
NPW = NPAD // NS

_mesh = plsc.VectorSubcoreMesh(
    core_axis_name="c", subcore_axis_name="s", num_cores=NC, num_subcores=NS
)


def _proj_body(nf_ref, w_ref, t_ref):
    t_ref[...] = jnp.dot(nf_ref[...], w_ref[...],
                         preferred_element_type=jnp.float32)


def _proj(nf, w):
    bn = 2000
    return pl.pallas_call(
        _proj_body,
        grid=(N // bn,),
        in_specs=[
            pl.BlockSpec((bn, D_NODE), lambda i: (i, 0)),
            pl.BlockSpec((D_NODE, 2 * H), lambda i: (0, 0)),
        ],
        out_specs=pl.BlockSpec((bn, 2 * H), lambda i: (i, 0)),
        out_shape=jax.ShapeDtypeStruct((N, 2 * H), jnp.float32),
    )(nf, w)


NBUF = 2
NGRP = NCHUNK // NBUF


@functools.partial(
    pl.kernel,
    out_type=jax.ShapeDtypeStruct((E, H), jnp.float32),
    mesh=_mesh,
    scratch_types=(
        [pltpu.VMEM((NCHUNK, CHUNK), jnp.int32)] * 2
        + [pltpu.VMEM((CHUNK, 2 * H), jnp.float32)] * (2 * NBUF)
        + [pltpu.VMEM((CHUNK, H), jnp.float32)] * NBUF
        + [pltpu.SemaphoreType.DMA] * (3 * NBUF)
    ),
)
def _gather_kernel(t_hbm, src_hbm, dst_hbm, g_hbm, *scratch):
    src_v, dst_v = scratch[0], scratch[1]
    rows_s = scratch[2:2 + NBUF]
    rows_d = scratch[2 + NBUF:2 + 2 * NBUF]
    gbuf = scratch[2 + 2 * NBUF:2 + 3 * NBUF]
    sem_s = scratch[2 + 3 * NBUF:2 + 4 * NBUF]
    sem_d = scratch[2 + 4 * NBUF:2 + 5 * NBUF]
    sem_w = scratch[2 + 5 * NBUF:2 + 6 * NBUF]

    cid = lax.axis_index("c")
    sid = lax.axis_index("s")
    wid = sid * NC + cid
    pltpu.sync_copy(src_hbm.at[wid], src_v)
    pltpu.sync_copy(dst_hbm.at[wid], dst_v)

    for b in range(NBUF):
        pltpu.async_copy(t_hbm.at[src_v.at[b]], rows_s[b], sem_s[b])
        pltpu.async_copy(t_hbm.at[dst_v.at[b]], rows_d[b], sem_d[b])

    def group(g, carry):
        for b in range(NBUF):
            c = g * NBUF + b
            pltpu.make_async_copy(t_hbm.at[pl.ds(0, CHUNK)],
                                  rows_s[b], sem_s[b]).wait()
            pltpu.make_async_copy(t_hbm.at[pl.ds(0, CHUNK)],
                                  rows_d[b], sem_d[b]).wait()

            @pl.when(g > 0)
            def _():
                pltpu.make_async_copy(g_hbm.at[pl.ds(0, CHUNK)],
                                      gbuf[b], sem_w[b]).wait()

            def add_row(i, acc, _b=b):
                for j in range(H // 16):
                    sl = pl.ds(j * 16, 16)
                    sl_hi = pl.ds(H + j * 16, 16)
                    gbuf[_b][i, sl] = rows_s[_b][i, sl] + rows_d[_b][i, sl_hi]
                return acc

            lax.fori_loop(0, CHUNK, add_row, 0)
            eoff = wid * EPW + c * CHUNK
            pltpu.async_copy(gbuf[b], g_hbm.at[pl.ds(eoff, CHUNK)], sem_w[b])

            @pl.when(g < NGRP - 1)
            def _():
                cn = c + NBUF
                pltpu.async_copy(t_hbm.at[src_v.at[cn]], rows_s[b], sem_s[b])
                pltpu.async_copy(t_hbm.at[dst_v.at[cn]], rows_d[b], sem_d[b])
        return carry

    lax.fori_loop(0, NGRP, group, 0)
    for b in range(NBUF):
        pltpu.make_async_copy(g_hbm.at[pl.ds(0, CHUNK)], gbuf[b], sem_w[b]).wait()


def _edge_mlp_body(ef_ref, g_ref, w1_ref, b1_ref, w2_ref, b2_ref,
                   w3_ref, b3_ref, out_ref):
    h1 = jnp.dot(ef_ref[...], w1_ref[...], preferred_element_type=jnp.float32)
    h1 = jnp.maximum(h1 + g_ref[...] + b1_ref[...], 0.0)
    h2 = jnp.dot(h1, w2_ref[...], preferred_element_type=jnp.float32)
    h2 = jnp.maximum(h2 + b2_ref[...], 0.0)
    h3 = jnp.dot(h2, w3_ref[...], preferred_element_type=jnp.float32)
    out_ref[...] = jnp.maximum(h3 + b3_ref[...], 0.0)


def _edge_mlp(ef, g, w1, b1, w2, b2, w3, b3):
    be = 3200
    wspec = lambda r, c: pl.BlockSpec((r, c), lambda i: (0, 0))
    return pl.pallas_call(
        _edge_mlp_body,
        grid=(E // be,),
        in_specs=[
            pl.BlockSpec((be, D_EDGE), lambda i: (i, 0)),
            pl.BlockSpec((be, H), lambda i: (i, 0)),
            wspec(D_EDGE, H), wspec(1, H),
            wspec(H, H), wspec(1, H),
            wspec(H, D_EOUT), wspec(1, D_EOUT),
        ],
        out_specs=pl.BlockSpec((be, D_EOUT), lambda i: (i, 0)),
        out_shape=jax.ShapeDtypeStruct((E, D_EOUT), jnp.float32),
    )(ef, g, w1, b1, w2, b2, w3, b3)


NBLK = 10
CPB = EPW * D_EOUT // (NBLK * 128)
FPB = CPB * 128
SLICE = NPAD * D_EOUT // NS


@functools.partial(
    pl.kernel,
    out_type=jax.ShapeDtypeStruct((NC, NPAD * D_EOUT), jnp.float32),
    mesh=_mesh,
    scratch_types=[
        pltpu.VMEM((FPB,), jnp.float32),
        pltpu.VMEM((CPB, 128), jnp.int32),
        pltpu.VMEM((SLICE,), jnp.float32),
        pltpu.VMEM_SHARED((NPAD * D_EOUT,), jnp.float32),
        pltpu.SemaphoreType.DMA,
    ],
)
def _scatter_kernel(uef_hbm, idx_hbm, out_hbm, data_v, idx_v, stage_v,
                    agg_sh, sem):
    cid = lax.axis_index("c")
    sid = lax.axis_index("s")
    wid = sid * NC + cid

    def zrow(i, carry):
        stage_v[pl.ds(i * 16, 16)] = jnp.zeros((16,), jnp.float32)
        return carry

    lax.fori_loop(0, SLICE // 16, zrow, 0)
    pltpu.sync_copy(stage_v, agg_sh.at[pl.ds(sid * SLICE, SLICE)])
    plsc.subcore_barrier()

    def blk(b, carry):
        foff = wid * (EPW * D_EOUT) + b * FPB
        pltpu.sync_copy(uef_hbm.at[pl.ds(foff, FPB)], data_v)
        pltpu.sync_copy(idx_hbm.at[wid].at[b], idx_v)

        def chnk(k, c2):
            pltpu.async_copy(data_v.at[pl.ds(k * 128, 128)],
                             agg_sh.at[idx_v.at[k]], sem, add=True)
            return c2

        lax.fori_loop(0, CPB, chnk, 0)
        pltpu.make_async_copy(uef_hbm.at[pl.ds(foff, FPB)], data_v, sem).wait()
        return carry

    lax.fori_loop(0, NBLK, blk, 0)
    plsc.subcore_barrier()

    pltpu.sync_copy(agg_sh.at[pl.ds(sid * SLICE, SLICE)], stage_v)
    pltpu.sync_copy(stage_v, out_hbm.at[cid].at[pl.ds(sid * SLICE, SLICE)])


def _node_mlp_body(pp_ref, nf_ref, wa_ref, wb_ref, b1_ref, w2_ref, b2_ref,
                   w3_ref, b3_ref, out_ref):
    agg = pp_ref[0] + pp_ref[1]
    h1 = (jnp.dot(agg, wa_ref[...], preferred_element_type=jnp.float32)
          + jnp.dot(nf_ref[...], wb_ref[...], preferred_element_type=jnp.float32))
    h1 = jnp.maximum(h1 + b1_ref[...], 0.0)
    h2 = jnp.dot(h1, w2_ref[...], preferred_element_type=jnp.float32)
    h2 = jnp.maximum(h2 + b2_ref[...], 0.0)
    h3 = jnp.dot(h2, w3_ref[...], preferred_element_type=jnp.float32)
    out_ref[...] = jnp.maximum(h3 + b3_ref[...], 0.0)


def _node_mlp(pp, nf, wa, wb, b1, w2, b2, w3, b3):
    bn = 2000
    wspec = lambda r, c: pl.BlockSpec((r, c), lambda i: (0, 0))
    return pl.pallas_call(
        _node_mlp_body,
        grid=(N // bn,),
        in_specs=[
            pl.BlockSpec((NC, bn, D_EOUT), lambda i: (0, i, 0)),
            pl.BlockSpec((bn, D_NODE), lambda i: (i, 0)),
            wspec(D_EOUT, H), wspec(D_NODE, H), wspec(1, H),
            wspec(H, H), wspec(1, H),
            wspec(H, D_NOUT), wspec(1, D_NOUT),
        ],
        out_specs=pl.BlockSpec((bn, D_NOUT), lambda i: (i, 0)),
        out_shape=jax.ShapeDtypeStruct((N, D_NOUT), jnp.float32),
    )(pp, nf, wa, wb, b1, w2, b2, w3, b3)


def kernel(nf, ef, edge_index, We1, be1, We2, be2, We3, be3,
           Wn1, bn1, Wn2, bn2, Wn3, bn3):
    src = edge_index[0].astype(jnp.int32).reshape(NW, NCHUNK, CHUNK)
    dst = edge_index[1].astype(jnp.int32).reshape(NW, NCHUNK, CHUNK)
    we1e = We1[:D_EDGE]
    we1sd = jnp.concatenate([We1[D_EDGE:D_EDGE + D_NODE],
                             We1[D_EDGE + D_NODE:]], axis=1)
    wn1a = Wn1[:D_EOUT]
    wn1b = Wn1[D_EOUT:]

    t = _proj(nf, we1sd)
    g = _gather_kernel(t, src, dst)
    u_ef = _edge_mlp(ef, g, we1e, be1.reshape(1, -1), We2, be2.reshape(1, -1),
                     We3, be3.reshape(1, -1))
    dst_flat = edge_index[1].astype(jnp.int32)
    idx_elem = ((dst_flat * D_EOUT)[:, None]
                + jnp.arange(D_EOUT, dtype=jnp.int32)[None, :])
    idx_elem = idx_elem.reshape(NW, NBLK, CPB, 128)
    uef_flat = u_ef.reshape(E * D_EOUT)
    partials = _scatter_kernel(uef_flat, idx_elem)
    partials = partials.reshape(NC, NPAD, D_EOUT)
    u_nf = _node_mlp(partials, nf, wn1a, wn1b, bn1.reshape(1, -1),
                     Wn2, bn2.reshape(1, -1), Wn3, bn3.reshape(1, -1))
    return (u_nf, u_ef)


# speedup vs baseline: 4.0967x; 1.0616x over previous
import functools

import jax
import jax.numpy as jnp
from jax import lax
from jax.experimental import pallas as pl
from jax.experimental.pallas import tpu as pltpu
from jax.experimental.pallas import tpu_sc as plsc

N = 10000
E = 320000
D_NODE = 128
D_EDGE = 16
D_EOUT = 16
D_NOUT = 128
H = 64

NC = 2
NS = 16
NW = NC * NS
EPW = E // NW
CHUNK = 40
NCHUNK = EPW // CHUNK
NPAD = 10240
NPW = NPAD // NS

_mesh = plsc.VectorSubcoreMesh(
    core_axis_name="c", subcore_axis_name="s", num_cores=NC, num_subcores=NS
)


def _proj_body(nf_ref, w_ref, t_ref):
    t_ref[...] = jnp.dot(nf_ref[...], w_ref[...],
                         preferred_element_type=jnp.float32)


def _proj(nf, w):
    bn = 2000
    return pl.pallas_call(
        _proj_body,
        grid=(N // bn,),
        in_specs=[
            pl.BlockSpec((bn, D_NODE), lambda i: (i, 0)),
            pl.BlockSpec((D_NODE, 2 * H), lambda i: (0, 0)),
        ],
        out_specs=pl.BlockSpec((bn, 2 * H), lambda i: (i, 0)),
        out_shape=jax.ShapeDtypeStruct((N, 2 * H), jnp.float32),
    )(nf, w)


NBUF = 2
NGRP = NCHUNK // NBUF


@functools.partial(
    pl.kernel,
    out_type=jax.ShapeDtypeStruct((E, H), jnp.float32),
    mesh=_mesh,
    scratch_types=(
        [pltpu.VMEM((NCHUNK, CHUNK), jnp.int32)] * 2
        + [pltpu.VMEM((CHUNK, 2 * H), jnp.float32)] * (2 * NBUF)
        + [pltpu.VMEM((CHUNK, H), jnp.float32)] * NBUF
        + [pltpu.SemaphoreType.DMA] * (3 * NBUF)
    ),
)
def _gather_kernel(t_hbm, src_hbm, dst_hbm, g_hbm, *scratch):
    src_v, dst_v = scratch[0], scratch[1]
    rows_s = scratch[2:2 + NBUF]
    rows_d = scratch[2 + NBUF:2 + 2 * NBUF]
    gbuf = scratch[2 + 2 * NBUF:2 + 3 * NBUF]
    sem_s = scratch[2 + 3 * NBUF:2 + 4 * NBUF]
    sem_d = scratch[2 + 4 * NBUF:2 + 5 * NBUF]
    sem_w = scratch[2 + 5 * NBUF:2 + 6 * NBUF]

    cid = lax.axis_index("c")
    sid = lax.axis_index("s")
    wid = sid * NC + cid
    pltpu.sync_copy(src_hbm.at[wid], src_v)
    pltpu.sync_copy(dst_hbm.at[wid], dst_v)

    for b in range(NBUF):
        pltpu.async_copy(t_hbm.at[src_v.at[b]], rows_s[b], sem_s[b])
        pltpu.async_copy(t_hbm.at[dst_v.at[b]], rows_d[b], sem_d[b])

    def group(g, carry):
        for b in range(NBUF):
            c = g * NBUF + b
            pltpu.make_async_copy(t_hbm.at[pl.ds(0, CHUNK)],
                                  rows_s[b], sem_s[b]).wait()
            pltpu.make_async_copy(t_hbm.at[pl.ds(0, CHUNK)],
                                  rows_d[b], sem_d[b]).wait()

            @pl.when(g > 0)
            def _():
                pltpu.make_async_copy(g_hbm.at[pl.ds(0, CHUNK)],
                                      gbuf[b], sem_w[b]).wait()

            def add_row(i, acc, _b=b):
                for j in range(H // 16):
                    sl = pl.ds(j * 16, 16)
                    sl_hi = pl.ds(H + j * 16, 16)
                    gbuf[_b][i, sl] = rows_s[_b][i, sl] + rows_d[_b][i, sl_hi]
                return acc

            lax.fori_loop(0, CHUNK, add_row, 0)
            eoff = wid * EPW + c * CHUNK
            pltpu.async_copy(gbuf[b], g_hbm.at[pl.ds(eoff, CHUNK)], sem_w[b])

            @pl.when(g < NGRP - 1)
            def _():
                cn = c + NBUF
                pltpu.async_copy(t_hbm.at[src_v.at[cn]], rows_s[b], sem_s[b])
                pltpu.async_copy(t_hbm.at[dst_v.at[cn]], rows_d[b], sem_d[b])
        return carry

    lax.fori_loop(0, NGRP, group, 0)
    for b in range(NBUF):
        pltpu.make_async_copy(g_hbm.at[pl.ds(0, CHUNK)], gbuf[b],
                              sem_w[b]).wait()


def _edge_mlp_body(ef_ref, g_ref, w1_ref, b1_ref, w2_ref, b2_ref,
                   w3_ref, b3_ref, out_ref, outp_ref):
    be = ef_ref.shape[0]
    h1 = jnp.dot(ef_ref[...].astype(jnp.bfloat16), w1_ref[...],
                 preferred_element_type=jnp.float32)
    h1 = jnp.maximum(h1 + g_ref[...] + b1_ref[...], 0.0)
    h2 = jnp.dot(h1.astype(jnp.bfloat16), w2_ref[...],
                 preferred_element_type=jnp.float32)
    h2 = jnp.maximum(h2 + b2_ref[...], 0.0)
    h3 = jnp.dot(h2.astype(jnp.bfloat16), w3_ref[...],
                 preferred_element_type=jnp.float32)
    h3 = jnp.maximum(h3 + b3_ref[...], 0.0)
    out_ref[...] = h3
    h3r = h3.reshape(be // 8, 8, D_EOUT)
    for j in range(8):
        outp_ref[:, j * D_EOUT:(j + 1) * D_EOUT] = h3r[:, j, :]


def _edge_mlp(ef, g, w1, b1, w2, b2, w3, b3):
    be = 3200
    wspec = lambda r, c: pl.BlockSpec((r, c), lambda i: (0, 0))
    return pl.pallas_call(
        _edge_mlp_body,
        grid=(E // be,),
        in_specs=[
            pl.BlockSpec((be, D_EDGE), lambda i: (i, 0)),
            pl.BlockSpec((be, H), lambda i: (i, 0)),
            wspec(D_EDGE, H), wspec(1, H),
            wspec(H, H), wspec(1, H),
            wspec(H, D_EOUT), wspec(1, D_EOUT),
        ],
        out_specs=[
            pl.BlockSpec((be, D_EOUT), lambda i: (i, 0)),
            pl.BlockSpec((be // 8, 8 * D_EOUT), lambda i: (i, 0)),
        ],
        out_shape=[
            jax.ShapeDtypeStruct((E, D_EOUT), jnp.float32),
            jax.ShapeDtypeStruct((E // 8, 8 * D_EOUT), jnp.float32),
        ],
    )(ef, g, w1, b1, w2, b2, w3, b3)


NBLK = 25
CPB = EPW * D_EOUT // (NBLK * 128)
FPB = CPB * 128
BPB = FPB // D_EOUT
SLICE = NPAD * D_EOUT // NS


@functools.partial(
    pl.kernel,
    out_type=jax.ShapeDtypeStruct((NC, NPAD * D_EOUT), jnp.float32),
    mesh=_mesh,
    scratch_types=[
        pltpu.VMEM((FPB,), jnp.float32),
        pltpu.VMEM((CPB, 128), jnp.int32),
        pltpu.VMEM((SLICE,), jnp.float32),
        pltpu.VMEM_SHARED((NPAD * D_EOUT,), jnp.float32),
        pltpu.SemaphoreType.DMA,
    ],
)
def _scatter_kernel(uef_hbm, idx_hbm, out_hbm, flat_v, idxf_v,
                    stage_v, agg_sh, sem):
    cid = lax.axis_index("c")
    sid = lax.axis_index("s")
    wid = sid * NC + cid

    def zrow(i, carry):
        stage_v[pl.ds(i * 16, 16)] = jnp.zeros((16,), jnp.float32)
        return carry

    lax.fori_loop(0, SLICE // 16, zrow, 0)
    pltpu.sync_copy(stage_v, agg_sh.at[pl.ds(sid * SLICE, SLICE)])
    plsc.subcore_barrier()

    def blk(b, carry):
        foff = wid * (EPW * D_EOUT) + b * FPB
        pltpu.sync_copy(uef_hbm.at[pl.ds(foff, FPB)], flat_v)
        pltpu.sync_copy(idx_hbm.at[wid].at[b], idxf_v)

        def chnk(k, c2):
            pltpu.async_copy(flat_v.at[pl.ds(k * 128, 128)],
                             agg_sh.at[idxf_v.at[k]], sem, add=True)
            return c2

        lax.fori_loop(0, CPB, chnk, 0)
        pltpu.make_async_copy(uef_hbm.at[pl.ds(foff, FPB)], flat_v, sem).wait()
        return carry

    lax.fori_loop(0, NBLK, blk, 0)
    plsc.subcore_barrier()

    pltpu.sync_copy(agg_sh.at[pl.ds(sid * SLICE, SLICE)], stage_v)
    pltpu.sync_copy(stage_v, out_hbm.at[cid].at[pl.ds(sid * SLICE, SLICE)])


def _node_mlp_body(pp_ref, nf_ref, wa_ref, wb_ref, b1_ref, w2_ref, b2_ref,
                   w3_ref, b3_ref, out_ref):
    agg = pp_ref[0] + pp_ref[1]
    h1 = (jnp.dot(agg, wa_ref[...], preferred_element_type=jnp.float32)
          + jnp.dot(nf_ref[...], wb_ref[...], preferred_element_type=jnp.float32))
    h1 = jnp.maximum(h1 + b1_ref[...], 0.0)
    h2 = jnp.dot(h1, w2_ref[...], preferred_element_type=jnp.float32)
    h2 = jnp.maximum(h2 + b2_ref[...], 0.0)
    h3 = jnp.dot(h2, w3_ref[...], preferred_element_type=jnp.float32)
    out_ref[...] = jnp.maximum(h3 + b3_ref[...], 0.0)


def _node_mlp(pp, nf, wa, wb, b1, w2, b2, w3, b3):
    bn = 2000
    wspec = lambda r, c: pl.BlockSpec((r, c), lambda i: (0, 0))
    return pl.pallas_call(
        _node_mlp_body,
        grid=(N // bn,),
        in_specs=[
            pl.BlockSpec((NC, bn, D_EOUT), lambda i: (0, i, 0)),
            pl.BlockSpec((bn, D_NODE), lambda i: (i, 0)),
            wspec(D_EOUT, H), wspec(D_NODE, H), wspec(1, H),
            wspec(H, H), wspec(1, H),
            wspec(H, D_NOUT), wspec(1, D_NOUT),
        ],
        out_specs=pl.BlockSpec((bn, D_NOUT), lambda i: (i, 0)),
        out_shape=jax.ShapeDtypeStruct((N, D_NOUT), jnp.float32),
    )(pp, nf, wa, wb, b1, w2, b2, w3, b3)


def kernel(nf, ef, edge_index, We1, be1, We2, be2, We3, be3,
           Wn1, bn1, Wn2, bn2, Wn3, bn3):
    src = edge_index[0].astype(jnp.int32).reshape(NW, NCHUNK, CHUNK)
    dst = edge_index[1].astype(jnp.int32).reshape(NW, NCHUNK, CHUNK)
    we1e = We1[:D_EDGE]
    we1sd = jnp.concatenate([We1[D_EDGE:D_EDGE + D_NODE],
                             We1[D_EDGE + D_NODE:]], axis=1)
    wn1a = Wn1[:D_EOUT]
    wn1b = Wn1[D_EOUT:]

    t = _proj(nf, we1sd)
    g = _gather_kernel(t, src, dst)
    u_ef, u_ef_pk = _edge_mlp(ef, g, we1e.astype(jnp.bfloat16),
                              be1.reshape(1, -1), We2.astype(jnp.bfloat16),
                              be2.reshape(1, -1), We3.astype(jnp.bfloat16),
                              be3.reshape(1, -1))
    uef_flat = u_ef_pk.reshape(E * D_EOUT)
    dst_flat = edge_index[1].astype(jnp.int32)
    idx_elem = ((dst_flat * D_EOUT)[:, None]
                + jnp.arange(D_EOUT, dtype=jnp.int32)[None, :])
    idx_elem = idx_elem.reshape(NW, NBLK, CPB, 128)
    partials = _scatter_kernel(uef_flat, idx_elem)
    partials = partials.reshape(NC, NPAD, D_EOUT)
    u_nf = _node_mlp(partials, nf, wn1a, wn1b, bn1.reshape(1, -1),
                     Wn2, bn2.reshape(1, -1), Wn3, bn3.reshape(1, -1))
    return (u_nf, u_ef)


# speedup vs baseline: 4.1299x; 1.0081x over previous
import functools

import jax
import jax.numpy as jnp
from jax import lax
from jax.experimental import pallas as pl
from jax.experimental.pallas import tpu as pltpu
from jax.experimental.pallas import tpu_sc as plsc

N = 10000
E = 320000
D_NODE = 128
D_EDGE = 16
D_EOUT = 16
D_NOUT = 128
H = 64

NC = 2
NS = 16
NW = NC * NS
EPW = E // NW
CHUNK = 40
NCHUNK = EPW // CHUNK
NPAD = 10240
NPW = NPAD // NS

_mesh = plsc.VectorSubcoreMesh(
    core_axis_name="c", subcore_axis_name="s", num_cores=NC, num_subcores=NS
)


def _proj_body(nf_ref, w_ref, t_ref):
    t_ref[...] = jnp.dot(nf_ref[...], w_ref[...],
                         preferred_element_type=jnp.float32)


def _proj(nf, w):
    bn = 2000
    return pl.pallas_call(
        _proj_body,
        grid=(N // bn,),
        in_specs=[
            pl.BlockSpec((bn, D_NODE), lambda i: (i, 0)),
            pl.BlockSpec((D_NODE, 2 * H), lambda i: (0, 0)),
        ],
        out_specs=pl.BlockSpec((bn, 2 * H), lambda i: (i, 0)),
        out_shape=jax.ShapeDtypeStruct((N, 2 * H), jnp.float32),
    )(nf, w)


NBUF = 2
NGRP = NCHUNK // NBUF


@functools.partial(
    pl.kernel,
    out_type=jax.ShapeDtypeStruct((E, H), jnp.float32),
    mesh=_mesh,
    scratch_types=(
        [pltpu.VMEM((NCHUNK, CHUNK), jnp.int32)] * 2
        + [pltpu.VMEM((CHUNK, 2 * H), jnp.float32)] * (2 * NBUF)
        + [pltpu.VMEM((CHUNK, H), jnp.float32)] * NBUF
        + [pltpu.SemaphoreType.DMA] * (3 * NBUF)
    ),
)
def _gather_kernel(t_hbm, src_hbm, dst_hbm, g_hbm, *scratch):
    src_v, dst_v = scratch[0], scratch[1]
    rows_s = scratch[2:2 + NBUF]
    rows_d = scratch[2 + NBUF:2 + 2 * NBUF]
    gbuf = scratch[2 + 2 * NBUF:2 + 3 * NBUF]
    sem_s = scratch[2 + 3 * NBUF:2 + 4 * NBUF]
    sem_d = scratch[2 + 4 * NBUF:2 + 5 * NBUF]
    sem_w = scratch[2 + 5 * NBUF:2 + 6 * NBUF]

    cid = lax.axis_index("c")
    sid = lax.axis_index("s")
    wid = sid * NC + cid
    pltpu.sync_copy(src_hbm.at[wid], src_v)
    pltpu.sync_copy(dst_hbm.at[wid], dst_v)

    for b in range(NBUF):
        pltpu.async_copy(t_hbm.at[src_v.at[b]], rows_s[b], sem_s[b])
        pltpu.async_copy(t_hbm.at[dst_v.at[b]], rows_d[b], sem_d[b])

    def group(g, carry):
        for b in range(NBUF):
            c = g * NBUF + b
            pltpu.make_async_copy(t_hbm.at[pl.ds(0, CHUNK)],
                                  rows_s[b], sem_s[b]).wait()
            pltpu.make_async_copy(t_hbm.at[pl.ds(0, CHUNK)],
                                  rows_d[b], sem_d[b]).wait()

            @pl.when(g > 0)
            def _():
                pltpu.make_async_copy(g_hbm.at[pl.ds(0, CHUNK)],
                                      gbuf[b], sem_w[b]).wait()

            def add_row(i, acc, _b=b):
                for r in range(2):
                    for j in range(H // 16):
                        sl = pl.ds(j * 16, 16)
                        sl_hi = pl.ds(H + j * 16, 16)
                        gbuf[_b][2 * i + r, sl] = (
                            rows_s[_b][2 * i + r, sl]
                            + rows_d[_b][2 * i + r, sl_hi])
                return acc

            lax.fori_loop(0, CHUNK // 2, add_row, 0)
            eoff = wid * EPW + c * CHUNK
            pltpu.async_copy(gbuf[b], g_hbm.at[pl.ds(eoff, CHUNK)], sem_w[b])

            @pl.when(g < NGRP - 1)
            def _():
                cn = c + NBUF
                pltpu.async_copy(t_hbm.at[src_v.at[cn]], rows_s[b], sem_s[b])
                pltpu.async_copy(t_hbm.at[dst_v.at[cn]], rows_d[b], sem_d[b])
        return carry

    lax.fori_loop(0, NGRP, group, 0)
    for b in range(NBUF):
        pltpu.make_async_copy(g_hbm.at[pl.ds(0, CHUNK)], gbuf[b],
                              sem_w[b]).wait()


def _edge_mlp_body(ef_ref, g_ref, w1_ref, b1_ref, w2_ref, b2_ref,
                   w3_ref, b3_ref, out_ref, outp_ref):
    be = ef_ref.shape[0]
    h1 = jnp.dot(ef_ref[...].astype(jnp.bfloat16), w1_ref[...],
                 preferred_element_type=jnp.float32)
    h1 = jnp.maximum(h1 + g_ref[...] + b1_ref[...], 0.0)
    h2 = jnp.dot(h1.astype(jnp.bfloat16), w2_ref[...],
                 preferred_element_type=jnp.float32)
    h2 = jnp.maximum(h2 + b2_ref[...], 0.0)
    h3 = jnp.dot(h2.astype(jnp.bfloat16), w3_ref[...],
                 preferred_element_type=jnp.float32)
    h3 = jnp.maximum(h3 + b3_ref[...], 0.0)
    out_ref[...] = h3
    h3r = h3.reshape(be // 8, 8, D_EOUT)
    for j in range(8):
        outp_ref[:, j * D_EOUT:(j + 1) * D_EOUT] = h3r[:, j, :]


def _edge_mlp(ef, g, w1, b1, w2, b2, w3, b3):
    be = 3200
    wspec = lambda r, c: pl.BlockSpec((r, c), lambda i: (0, 0))
    return pl.pallas_call(
        _edge_mlp_body,
        grid=(E // be,),
        in_specs=[
            pl.BlockSpec((be, D_EDGE), lambda i: (i, 0)),
            pl.BlockSpec((be, H), lambda i: (i, 0)),
            wspec(D_EDGE, H), wspec(1, H),
            wspec(H, H), wspec(1, H),
            wspec(H, D_EOUT), wspec(1, D_EOUT),
        ],
        out_specs=[
            pl.BlockSpec((be, D_EOUT), lambda i: (i, 0)),
            pl.BlockSpec((be // 8, 8 * D_EOUT), lambda i: (i, 0)),
        ],
        out_shape=[
            jax.ShapeDtypeStruct((E, D_EOUT), jnp.float32),
            jax.ShapeDtypeStruct((E // 8, 8 * D_EOUT), jnp.float32),
        ],
    )(ef, g, w1, b1, w2, b2, w3, b3)


NBLK = 10
CPB = EPW * D_EOUT // (NBLK * 128)
FPB = CPB * 128
BPB = FPB // D_EOUT
SLICE = NPAD * D_EOUT // NS


@functools.partial(
    pl.kernel,
    out_type=jax.ShapeDtypeStruct((NC, NPAD * D_EOUT), jnp.float32),
    mesh=_mesh,
    scratch_types=[
        pltpu.VMEM((FPB,), jnp.float32),
        pltpu.VMEM((CPB, 128), jnp.int32),
        pltpu.VMEM((SLICE,), jnp.float32),
        pltpu.VMEM_SHARED((NPAD * D_EOUT,), jnp.float32),
        pltpu.SemaphoreType.DMA,
    ],
)
def _scatter_kernel(uef_hbm, idx_hbm, out_hbm, flat_v, idxf_v,
                    stage_v, agg_sh, sem):
    cid = lax.axis_index("c")
    sid = lax.axis_index("s")
    wid = sid * NC + cid

    def zrow(i, carry):
        stage_v[pl.ds(i * 16, 16)] = jnp.zeros((16,), jnp.float32)
        return carry

    lax.fori_loop(0, SLICE // 16, zrow, 0)
    pltpu.sync_copy(stage_v, agg_sh.at[pl.ds(sid * SLICE, SLICE)])
    plsc.subcore_barrier()

    def blk(b, carry):
        foff = wid * (EPW * D_EOUT) + b * FPB
        pltpu.sync_copy(uef_hbm.at[pl.ds(foff, FPB)], flat_v)
        pltpu.sync_copy(idx_hbm.at[wid].at[b], idxf_v)

        def chnk(k, c2):
            pltpu.async_copy(flat_v.at[pl.ds(k * 128, 128)],
                             agg_sh.at[idxf_v.at[k]], sem, add=True)
            return c2

        lax.fori_loop(0, CPB, chnk, 0)
        pltpu.make_async_copy(uef_hbm.at[pl.ds(foff, FPB)], flat_v, sem).wait()
        return carry

    lax.fori_loop(0, NBLK, blk, 0)
    plsc.subcore_barrier()

    pltpu.sync_copy(agg_sh.at[pl.ds(sid * SLICE, SLICE)], stage_v)
    pltpu.sync_copy(stage_v, out_hbm.at[cid].at[pl.ds(sid * SLICE, SLICE)])


def _node_mlp_body(pp_ref, nf_ref, wa_ref, wb_ref, b1_ref, w2_ref, b2_ref,
                   w3_ref, b3_ref, out_ref):
    agg = pp_ref[0] + pp_ref[1]
    h1 = (jnp.dot(agg, wa_ref[...], preferred_element_type=jnp.float32)
          + jnp.dot(nf_ref[...], wb_ref[...], preferred_element_type=jnp.float32))
    h1 = jnp.maximum(h1 + b1_ref[...], 0.0)
    h2 = jnp.dot(h1, w2_ref[...], preferred_element_type=jnp.float32)
    h2 = jnp.maximum(h2 + b2_ref[...], 0.0)
    h3 = jnp.dot(h2, w3_ref[...], preferred_element_type=jnp.float32)
    out_ref[...] = jnp.maximum(h3 + b3_ref[...], 0.0)


def _node_mlp(pp, nf, wa, wb, b1, w2, b2, w3, b3):
    bn = 2000
    wspec = lambda r, c: pl.BlockSpec((r, c), lambda i: (0, 0))
    return pl.pallas_call(
        _node_mlp_body,
        grid=(N // bn,),
        in_specs=[
            pl.BlockSpec((NC, bn, D_EOUT), lambda i: (0, i, 0)),
            pl.BlockSpec((bn, D_NODE), lambda i: (i, 0)),
            wspec(D_EOUT, H), wspec(D_NODE, H), wspec(1, H),
            wspec(H, H), wspec(1, H),
            wspec(H, D_NOUT), wspec(1, D_NOUT),
        ],
        out_specs=pl.BlockSpec((bn, D_NOUT), lambda i: (i, 0)),
        out_shape=jax.ShapeDtypeStruct((N, D_NOUT), jnp.float32),
    )(pp, nf, wa, wb, b1, w2, b2, w3, b3)


def kernel(nf, ef, edge_index, We1, be1, We2, be2, We3, be3,
           Wn1, bn1, Wn2, bn2, Wn3, bn3):
    src = edge_index[0].astype(jnp.int32).reshape(NW, NCHUNK, CHUNK)
    dst = edge_index[1].astype(jnp.int32).reshape(NW, NCHUNK, CHUNK)
    we1e = We1[:D_EDGE]
    we1sd = jnp.concatenate([We1[D_EDGE:D_EDGE + D_NODE],
                             We1[D_EDGE + D_NODE:]], axis=1)
    wn1a = Wn1[:D_EOUT]
    wn1b = Wn1[D_EOUT:]

    t = _proj(nf, we1sd)
    g = _gather_kernel(t, src, dst)
    u_ef, u_ef_pk = _edge_mlp(ef, g, we1e.astype(jnp.bfloat16),
                              be1.reshape(1, -1), We2.astype(jnp.bfloat16),
                              be2.reshape(1, -1), We3.astype(jnp.bfloat16),
                              be3.reshape(1, -1))
    uef_flat = u_ef_pk.reshape(E * D_EOUT)
    dst_flat = edge_index[1].astype(jnp.int32)
    idx_elem = (dst_flat.reshape(E // 8, 8, 1) * D_EOUT
                + jnp.arange(D_EOUT, dtype=jnp.int32).reshape(1, 1, D_EOUT))
    idx_elem = idx_elem.reshape(NW, NBLK, CPB, 128)
    partials = _scatter_kernel(uef_flat, idx_elem)
    partials = partials.reshape(NC, NPAD, D_EOUT)
    u_nf = _node_mlp(partials, nf, wn1a, wn1b, bn1.reshape(1, -1),
                     Wn2, bn2.reshape(1, -1), Wn3, bn3.reshape(1, -1))
    return (u_nf, u_ef)
